# baseline (speedup 1.0000x reference)
import functools

import jax
import jax.numpy as jnp
from jax import lax
from jax.experimental import pallas as pl
from jax.experimental.pallas import tpu as pltpu
from jax.experimental.pallas import tpu_sc as plsc

BATCH = 16384
DIM = 64
NUM_CORES = 2
NUM_SUBCORES = 16
NUM_WORKERS = NUM_CORES * NUM_SUBCORES
ROWS_PER_WORKER = BATCH // NUM_WORKERS
CHUNK = 128
NCHUNK = ROWS_PER_WORKER // CHUNK


def _gather_body(table_hbm, lab_hbm, out_hbm, idx_v, rows_v, sem):
    wid = lax.axis_index("s") * NUM_CORES + lax.axis_index("c")
    base = wid * NCHUNK
    pltpu.sync_copy(lab_hbm.at[pl.ds(base, NCHUNK)], idx_v)
    copies = [
        pltpu.async_copy(table_hbm.at[idx_v.at[j]], rows_v.at[j], sem)
        for j in range(NCHUNK)
    ]
    for c in copies:
        c.wait()
    pltpu.sync_copy(rows_v, out_hbm.at[pl.ds(base, NCHUNK)])


@jax.jit
def _sc_gather(table, lab2d):
    mesh = plsc.VectorSubcoreMesh(core_axis_name="c", subcore_axis_name="s")
    f = functools.partial(
        pl.kernel,
        out_type=jax.ShapeDtypeStruct((BATCH // CHUNK, CHUNK, DIM), jnp.float32),
        mesh=mesh,
        scratch_types=[
            pltpu.VMEM((NCHUNK, CHUNK), jnp.int32),
            pltpu.VMEM((NCHUNK, CHUNK, DIM), jnp.float32),
            pltpu.SemaphoreType.DMA,
        ],
        compiler_params=pltpu.CompilerParams(use_tc_tiling_on_sc=False),
    )(_gather_body)
    return f(table, lab2d)


BLK = 4096
NBLK = BATCH // BLK


def _loss_body(x_ref, d_ref, w_ref, o_ref):
    i = pl.program_id(0)
    xb = x_ref[...]
    db = d_ref[...]
    dot = jnp.sum(db * xb, axis=1, keepdims=True)
    na = jnp.maximum(jnp.sqrt(jnp.sum(db * db, axis=1, keepdims=True)), 1e-8)
    nb = jnp.maximum(jnp.sqrt(jnp.sum(xb * xb, axis=1, keepdims=True)), 1e-8)
    part = jnp.sum(dot / (na * nb) * w_ref[...])

    @pl.when(i == 0)
    def _init():
        o_ref[...] = jnp.zeros_like(o_ref)

    o_ref[...] += part

    @pl.when(i == NBLK - 1)
    def _fin():
        o_ref[...] = o_ref[...] * (-1.0 / BATCH)


@jax.jit
def _tc_loss(x, dense, w2):
    return pl.pallas_call(
        _loss_body,
        grid=(NBLK,),
        in_specs=[
            pl.BlockSpec((BLK, DIM), lambda i: (i, 0)),
            pl.BlockSpec((BLK, DIM), lambda i: (i, 0)),
            pl.BlockSpec((BLK, 1), lambda i: (i, 0)),
        ],
        out_specs=pl.BlockSpec((1, 1), lambda i: (0, 0)),
        out_shape=jax.ShapeDtypeStruct((1, 1), jnp.float32),
    )(x, dense, w2)


def kernel(x, label, weight, embedding_matrix):
    lab2d = label.astype(jnp.int32).reshape(BATCH // CHUNK, CHUNK)
    dense = _sc_gather(embedding_matrix, lab2d).reshape(BATCH, DIM)
    out = _tc_loss(x, dense, weight.reshape(BATCH, 1))
    return out[0, 0]


# speedup vs baseline: 1.5132x; 1.5132x over previous
import functools

import jax
import jax.numpy as jnp
from jax import lax
from jax.experimental import pallas as pl
from jax.experimental.pallas import tpu as pltpu
from jax.experimental.pallas import tpu_sc as plsc

BATCH = 16384
DIM = 64
PAD = 128
NUM_CORES = 2
NUM_SUBCORES = 16
NUM_WORKERS = NUM_CORES * NUM_SUBCORES
ROWS_PER_WORKER = BATCH // NUM_WORKERS
SUBL = 8
NGRP = 100000 // SUBL
G = 32
NCHUNK = ROWS_PER_WORKER // G


def _gather_body(table_hbm, lab_hbm, out_hbm, lab_sh, lab_s, rows_v, sem):
    cid = lax.axis_index("c")
    sub = lax.axis_index("s")
    wid = sub * NUM_CORES + cid
    base = wid * ROWS_PER_WORKER
    pltpu.sync_copy(lab_hbm.at[pl.ds(base, ROWS_PER_WORKER)], lab_sh.at[sub])
    pltpu.sync_copy(lab_sh.at[sub], lab_s)

    def chunk(ci, carry):
        copies = []
        for j in range(G):
            l = lab_s[ci * G + j]
            copies.append(
                pltpu.async_copy(
                    table_hbm.at[lax.shift_right_logical(l, 3),
                                 lax.bitwise_and(l, 7)],
                    rows_v.at[ci * G + j, pl.ds(0, DIM)],
                    sem,
                )
            )
        for c in copies:
            c.wait()
        return carry

    lax.fori_loop(0, NCHUNK, chunk, 0)
    pltpu.sync_copy(rows_v, out_hbm.at[pl.ds(base, ROWS_PER_WORKER)])


@jax.jit
def _sc_gather(table3d, lab):
    mesh = plsc.VectorSubcoreMesh(core_axis_name="c", subcore_axis_name="s")
    f = functools.partial(
        pl.kernel,
        out_type=jax.ShapeDtypeStruct((BATCH, PAD), jnp.float32),
        mesh=mesh,
        scratch_types=[
            pltpu.VMEM_SHARED((NUM_SUBCORES, ROWS_PER_WORKER), jnp.int32),
            pltpu.SMEM((ROWS_PER_WORKER,), jnp.int32),
            pltpu.VMEM((ROWS_PER_WORKER, PAD), jnp.float32),
            pltpu.SemaphoreType.DMA,
        ],
        compiler_params=pltpu.CompilerParams(needs_layout_passes=False),
    )(_gather_body)
    return f(table3d, lab)


BLK = 4096
NBLK = BATCH // BLK


def _loss_body(x_ref, d_ref, w_ref, o_ref):
    i = pl.program_id(0)
    xb = x_ref[...]
    db = d_ref[:, :DIM]
    dot = jnp.sum(db * xb, axis=1, keepdims=True)
    na = jnp.maximum(jnp.sqrt(jnp.sum(db * db, axis=1, keepdims=True)), 1e-8)
    nb = jnp.maximum(jnp.sqrt(jnp.sum(xb * xb, axis=1, keepdims=True)), 1e-8)
    part = jnp.sum(dot / (na * nb) * w_ref[...])

    @pl.when(i == 0)
    def _init():
        o_ref[...] = jnp.zeros_like(o_ref)

    o_ref[...] += part

    @pl.when(i == NBLK - 1)
    def _fin():
        o_ref[...] = o_ref[...] * (-1.0 / BATCH)


@jax.jit
def _tc_loss(x, dense, w2):
    return pl.pallas_call(
        _loss_body,
        grid=(NBLK,),
        in_specs=[
            pl.BlockSpec((BLK, DIM), lambda i: (i, 0)),
            pl.BlockSpec((BLK, PAD), lambda i: (i, 0)),
            pl.BlockSpec((BLK, 1), lambda i: (i, 0)),
        ],
        out_specs=pl.BlockSpec((1, 1), lambda i: (0, 0)),
        out_shape=jax.ShapeDtypeStruct((1, 1), jnp.float32),
    )(x, dense, w2)


def kernel(x, label, weight, embedding_matrix):
    lab = label.astype(jnp.int32)
    table3d = embedding_matrix.reshape(NGRP, SUBL, DIM)
    dense = _sc_gather(table3d, lab)
    out = _tc_loss(x, dense, weight.reshape(BATCH, 1))
    return out[0, 0]
